# baseline (speedup 1.0000x reference)
import functools

import jax
import jax.numpy as jnp
from jax import lax
from jax.experimental import pallas as pl
from jax.experimental.pallas import tpu as pltpu
from jax.experimental.pallas import tpu_sc as plsc

B = 4096
D = 32
NB = 16
NR = 64
NC = 2
NS = 16
NW = NC * NS
BPW = B // NW

_mesh = plsc.VectorSubcoreMesh(core_axis_name="c", subcore_axis_name="s")
_PIB = lax.GatherScatterMode.PROMISE_IN_BOUNDS


def _perm(x, idx):
    return jnp.take_along_axis(x, idx, axis=0, mode=_PIB)


def _lane_max(x):
    i = lax.iota(jnp.int32, 16)
    for sh in (1, 2, 4, 8):
        x = jnp.maximum(x, _perm(x, i ^ sh))
    return x


def _lane_sum(x):
    i = lax.iota(jnp.int32, 16)
    for sh in (1, 2, 4, 8):
        x = x + _perm(x, i ^ sh)
    return x


def _gather64(sb, r):
    out = jnp.zeros((16,), jnp.float32)
    for c in range(4):
        idx = r - c * 16
        m = (idx >= 0) & (idx < 16)
        idxc = jnp.clip(idx, 0, 15)
        out = jnp.where(m, _perm(sb[c], idxc), out)
    return out


@functools.partial(
    pl.kernel,
    out_type=[
        jax.ShapeDtypeStruct((B, D), jnp.float32),
        jax.ShapeDtypeStruct((B, D), jnp.float32),
        jax.ShapeDtypeStruct((B * NB, D), jnp.float32),
        jax.ShapeDtypeStruct((B * NB, D), jnp.float32),
        jax.ShapeDtypeStruct((B, NB), jnp.float32),
        jax.ShapeDtypeStruct((B, NB * NB), jnp.float32),
    ],
    mesh=_mesh,
    compiler_params=pltpu.CompilerParams(use_tc_tiling_on_sc=False),
    scratch_types=[
        pltpu.VMEM((BPW,), jnp.int32),
        pltpu.VMEM((BPW, D), jnp.float32),
        pltpu.VMEM((BPW, D), jnp.float32),
        pltpu.VMEM((NB, BPW), jnp.int32),
        pltpu.VMEM((2, BPW, D), jnp.float32),
        pltpu.VMEM((D, NR), jnp.float32),
        pltpu.VMEM((BPW, NB), jnp.int32),
        pltpu.VMEM((BPW, NB), jnp.float32),
        pltpu.VMEM((2 * BPW, BPW), jnp.int32),
        pltpu.VMEM((BPW, NB * NB), jnp.int32),
        pltpu.VMEM((4, NB * NB, D), jnp.float32),
        pltpu.VMEM((2, NB, D), jnp.float32),
        pltpu.VMEM((2, NB * NB), jnp.float32),
        pltpu.SemaphoreType.DMA,
        pltpu.SemaphoreType.DMA,
        pltpu.SemaphoreType.DMA,
        pltpu.SemaphoreType.DMA,
        pltpu.SemaphoreType.DMA,
        pltpu.SemaphoreType.DMA,
        pltpu.SemaphoreType.DMA,
        pltpu.SemaphoreType.DMA,
        pltpu.SemaphoreType.DMA,
        pltpu.SemaphoreType.DMA,
        pltpu.SemaphoreType.DMA,
    ],
)
def _sc_gather(usr_w, ent_w, relT, u, v, e1r, e2r, r2, r1,
               user_out, ev0_out, ev1_out, agg2_out, p1_out, ew_out,
               idx_v, user_rows, ev0_rows, e1_v, rowbuf, relT_v, r1_v,
               p1_v, e2_v, r2_v, rows_v, agg_v, ws_v,
               sem, semG0, semG1, semO0, semO1, semM0, semM1, semM2, semM3,
               semA0, semA1):
    wid = lax.axis_index("s") * NC + lax.axis_index("c")
    base = wid * BPW
    fbase = wid * BPW * NB
    semG = (semG0, semG1)
    semO = (semO0, semO1)
    semM = (semM0, semM1, semM2, semM3)
    semA = (semA0, semA1)

    pltpu.sync_copy(u.at[pl.ds(base, BPW)], idx_v)
    pltpu.async_copy(usr_w.at[idx_v], user_rows, sem).wait()
    pltpu.sync_copy(user_rows, user_out.at[pl.ds(base, BPW)])

    pltpu.sync_copy(v.at[pl.ds(base, BPW)], idx_v)
    pltpu.async_copy(ent_w.at[idx_v], ev0_rows, sem).wait()
    pltpu.sync_copy(ev0_rows, ev0_out.at[pl.ds(base, BPW)])

    pltpu.sync_copy(e1r.at[pl.ds(wid * NB, NB)], e1_v)

    def ev1_issue(c, buf):
        pltpu.async_copy(ent_w.at[e1_v.at[c]], rowbuf.at[buf], semG[buf])

    def ev1_out_copy(c, buf):
        return pltpu.make_async_copy(
            rowbuf.at[buf], ev1_out.at[pl.ds(fbase + c * BPW, BPW)], semO[buf])

    ev1_issue(0, 0)
    for c in range(NB):
        buf = c & 1
        if c + 1 < NB:
            if c - 1 >= 0:
                ev1_out_copy(c - 1, 1 - buf).wait()
            ev1_issue(c + 1, 1 - buf)
        pltpu.make_async_copy(ent_w.at[e1_v.at[c]], rowbuf.at[buf],
                              semG[buf]).wait()
        ev1_out_copy(c, buf).start()
    ev1_out_copy(NB - 2, (NB - 2) & 1).wait()
    ev1_out_copy(NB - 1, (NB - 1) & 1).wait()

    pltpu.sync_copy(relT, relT_v)
    pltpu.sync_copy(r1.at[pl.ds(base, BPW)], r1_v)
    pltpu.sync_copy(e2r.at[pl.ds(wid * 2 * BPW, 2 * BPW)], e2_v)
    pltpu.sync_copy(r2.at[pl.ds(base, BPW)], r2_v)

    def main_issue(b, buf):
        pltpu.async_copy(ent_w.at[e2_v.at[2 * b]],
                         rows_v.at[buf, pl.ds(0, BPW)], semM[buf])
        pltpu.async_copy(ent_w.at[e2_v.at[2 * b + 1]],
                         rows_v.at[buf, pl.ds(BPW, BPW)], semM[buf])

    def main_drain(b, buf):
        pltpu.make_async_copy(ent_w.at[e2_v.at[2 * b]],
                              rows_v.at[buf, pl.ds(0, BPW)], semM[buf]).wait()
        pltpu.make_async_copy(ent_w.at[e2_v.at[2 * b + 1]],
                              rows_v.at[buf, pl.ds(BPW, BPW)], semM[buf]).wait()

    def agg_copy(b, buf):
        return pltpu.make_async_copy(
            agg_v.at[buf], agg2_out.at[pl.ds(fbase + b * NB, NB)], semA[buf])

    def ew_copy(b, buf):
        return pltpu.make_async_copy(
            ws_v.at[buf], ew_out.at[base + b], semA[buf])

    def compute_b(b, buf, abuf):
        ur = [user_rows[b, 0:16], user_rows[b, 16:32]]
        sb = []
        for rc in range(4):
            accs = [jnp.zeros((16,), jnp.float32) for _ in range(4)]
            for dd in range(D):
                accs[dd % 4] = accs[dd % 4] + (
                    ur[dd // 16][dd % 16] * relT_v[dd, rc * 16:(rc + 1) * 16])
            sb.append((accs[0] + accs[1]) + (accs[2] + accs[3]))
        p1_v[b, :] = jnp.exp(_gather64(sb, r1_v[b, :]))
        ws = []
        for n in range(NB):
            e = jnp.exp(_gather64(sb, r2_v[b, n * 16:(n + 1) * 16]))
            ws_v[abuf, n * 16:(n + 1) * 16] = e
            ws.append(e)
        for n in range(NB):
            e = ws[n]
            a0s = [jnp.zeros((16,), jnp.float32) for _ in range(4)]
            a1s = [jnp.zeros((16,), jnp.float32) for _ in range(4)]
            for k in range(NB):
                w = e[k]
                a0s[k % 4] = a0s[k % 4] + w * rows_v[buf, n * NB + k, 0:16]
                a1s[k % 4] = a1s[k % 4] + w * rows_v[buf, n * NB + k, 16:32]
            agg_v[abuf, n, 0:16] = (a0s[0] + a0s[1]) + (a0s[2] + a0s[3])
            agg_v[abuf, n, 16:32] = (a1s[0] + a1s[1]) + (a1s[2] + a1s[3])
        agg_copy(b, abuf).start()
        ew_copy(b, abuf).start()

    main_issue(0, 0)
    main_issue(1, 1)

    def main_body(i, carry):
        b0 = 4 * i

        def aggwait(x, abuf, guarded):
            if guarded:
                @pl.when(i > 0)
                def _():
                    agg_copy(x, abuf).wait()
                    ew_copy(x, abuf).wait()
            else:
                agg_copy(x, abuf).wait()
                ew_copy(x, abuf).wait()

        main_issue(b0 + 2, 2)
        main_issue(b0 + 3, 3)
        main_drain(b0, 0)
        aggwait(b0 - 2, 0, True)
        compute_b(b0, 0, 0)
        main_drain(b0 + 1, 1)
        aggwait(b0 - 1, 1, True)
        compute_b(b0 + 1, 1, 1)

        @pl.when(i < BPW // 4 - 1)
        def _():
            main_issue(b0 + 4, 0)
            main_issue(b0 + 5, 1)

        main_drain(b0 + 2, 2)
        aggwait(b0, 0, False)
        compute_b(b0 + 2, 2, 0)
        main_drain(b0 + 3, 3)
        aggwait(b0 + 1, 1, False)
        compute_b(b0 + 3, 3, 1)
        return carry

    lax.fori_loop(0, BPW // 4, main_body, 0)
    agg_copy(BPW - 2, 0).wait()
    ew_copy(BPW - 2, 0).wait()
    agg_copy(BPW - 1, 1).wait()
    ew_copy(BPW - 1, 1).wait()
    pltpu.sync_copy(p1_v, p1_out.at[pl.ds(base, BPW)])


CB = 8192


def _pack_body(src_ref, out_ref):
    x = src_ref[...]
    y = jnp.concatenate(
        [x[:, q * (CB // 4):(q + 1) * (CB // 4)] for q in range(4)], axis=0)
    out_ref[...] = y.T


def _pack_table(tT):
    n = tT.shape[1]
    grid = (n + CB - 1) // CB
    out = pl.pallas_call(
        _pack_body,
        grid=(grid,),
        in_specs=[pl.BlockSpec((D, CB), lambda i: (0, i))],
        out_specs=pl.BlockSpec((CB // 4, 128), lambda i: (i, 0)),
        out_shape=jax.ShapeDtypeStruct((grid * (CB // 4), 128), jnp.float32),
    )(tT)
    return out.reshape(grid * CB, D)


_CBQ = CB // 4
_CBSH = _CBQ.bit_length() - 1


def _rowmap(i):
    return (i & ~(CB - 1)) + 4 * (i & (_CBQ - 1)) + ((i >> _CBSH) & 3)


BB = 256


def _tc_body(user_ref, ev0_ref, ev1_ref, agg2_ref, p1_ref, ew_ref, W_ref,
             b_ref, out_ref):
    user = user_ref[...]
    ev0 = ev0_ref[...]
    ev1 = ev1_ref[...]
    agg2 = agg2_ref[...]
    p1r = p1_ref[...]
    ew = ew_ref[...]
    W = W_ref[...]
    bias = b_ref[...]
    p1 = p1r / jnp.sum(p1r, axis=1, keepdims=True)
    z = jnp.sum(ew.reshape(BB, NB, NB), axis=2)
    rz = 1.0 / z
    agg1 = jnp.zeros((BB, D), jnp.float32)
    itemagg = jnp.zeros((BB, D), jnp.float32)
    for k in range(NB):
        evk = ev1[:, k * D:(k + 1) * D]
        pk = p1[:, k:k + 1]
        agg1 = agg1 + pk * evk
        x1 = evk + agg2[:, k * D:(k + 1) * D] * rz[:, k:k + 1]
        h1k = jax.nn.sigmoid(
            jnp.dot(x1, W, preferred_element_type=jnp.float32) + bias)
        itemagg = itemagg + pk * h1k
    h0 = jax.nn.sigmoid(
        jnp.dot(ev0 + agg1, W, preferred_element_type=jnp.float32) + bias)
    item = jnp.tanh(
        jnp.dot(h0 + itemagg, W, preferred_element_type=jnp.float32) + bias)
    out_ref[...] = jax.nn.sigmoid(jnp.sum(user * item, axis=1)).reshape(1, 1, BB)


def _tc_tail(user, ev0, ev1f, agg2f, p1, ew, W, b2):
    grid = B // BB
    out = pl.pallas_call(
        _tc_body,
        grid=(grid,),
        in_specs=[
            pl.BlockSpec((BB, D), lambda i: (i, 0)),
            pl.BlockSpec((BB, D), lambda i: (i, 0)),
            pl.BlockSpec((BB, NB * D), lambda i: (i, 0)),
            pl.BlockSpec((BB, NB * D), lambda i: (i, 0)),
            pl.BlockSpec((BB, NB), lambda i: (i, 0)),
            pl.BlockSpec((BB, NB * NB), lambda i: (i, 0)),
            pl.BlockSpec((D, D), lambda i: (0, 0)),
            pl.BlockSpec((1, D), lambda i: (0, 0)),
        ],
        out_specs=pl.BlockSpec((1, 1, BB), lambda i: (i, 0, 0)),
        out_shape=jax.ShapeDtypeStruct((grid, 1, BB), jnp.float32),
    )(user, ev0, ev1f, agg2f, p1, ew, W, b2)
    return out.reshape(B)


def kernel(usr_w, ent_w, rel_w, W, b, u, v, e1, e2, r1, r2):
    ent_g = _pack_table(ent_w.T)
    usr_g = _pack_table(usr_w.T)
    relT = rel_w.T
    e1r = _rowmap(e1).reshape(B * NB // BPW, BPW)
    e2r = _rowmap(e2).reshape(2 * B, BPW)
    user, ev0, ev1f, agg2f, p1, ew = _sc_gather(
        usr_g, ent_g, relT, _rowmap(u), _rowmap(v), e1r, e2r, r2, r1)
    return _tc_tail(user, ev0, ev1f.reshape(B, NB * D),
                    agg2f.reshape(B, NB * D), p1, ew, W, b.reshape(1, D))


# speedup vs baseline: 1.1602x; 1.1602x over previous
import functools

import jax
import jax.numpy as jnp
from jax import lax
from jax.experimental import pallas as pl
from jax.experimental.pallas import tpu as pltpu
from jax.experimental.pallas import tpu_sc as plsc

B = 4096
D = 32
NB = 16
NR = 64
NC = 2
NS = 16
NW = NC * NS
BPW = B // NW

_mesh = plsc.VectorSubcoreMesh(core_axis_name="c", subcore_axis_name="s")
_PIB = lax.GatherScatterMode.PROMISE_IN_BOUNDS


def _perm(x, idx):
    return jnp.take_along_axis(x, idx, axis=0, mode=_PIB)


def _lane_max(x):
    i = lax.iota(jnp.int32, 16)
    for sh in (1, 2, 4, 8):
        x = jnp.maximum(x, _perm(x, i ^ sh))
    return x


def _lane_sum(x):
    i = lax.iota(jnp.int32, 16)
    for sh in (1, 2, 4, 8):
        x = x + _perm(x, i ^ sh)
    return x


def _gather64(sb, r):
    out = jnp.zeros((16,), jnp.float32)
    for c in range(4):
        idx = r - c * 16
        m = (idx >= 0) & (idx < 16)
        idxc = jnp.clip(idx, 0, 15)
        out = jnp.where(m, _perm(sb[c], idxc), out)
    return out


@functools.partial(
    pl.kernel,
    out_type=[
        jax.ShapeDtypeStruct((B, D), jnp.float32),
        jax.ShapeDtypeStruct((B, D), jnp.float32),
        jax.ShapeDtypeStruct((B * NB, D), jnp.float32),
        jax.ShapeDtypeStruct((B * NB, D), jnp.float32),
        jax.ShapeDtypeStruct((B, NB), jnp.float32),
        jax.ShapeDtypeStruct((B, NB * NB), jnp.float32),
    ],
    mesh=_mesh,
    compiler_params=pltpu.CompilerParams(use_tc_tiling_on_sc=False),
    scratch_types=[
        pltpu.VMEM((BPW,), jnp.int32),
        pltpu.VMEM((BPW, D), jnp.float32),
        pltpu.VMEM((BPW, D), jnp.float32),
        pltpu.VMEM((NB, BPW), jnp.int32),
        pltpu.VMEM((2, BPW, D), jnp.float32),
        pltpu.VMEM((D, NR), jnp.float32),
        pltpu.VMEM((BPW, NB), jnp.int32),
        pltpu.VMEM((BPW, NB), jnp.float32),
        pltpu.VMEM((2 * BPW, BPW), jnp.int32),
        pltpu.VMEM((BPW, NB * NB), jnp.int32),
        pltpu.VMEM((2, NB * NB, D), jnp.float32),
        pltpu.VMEM((2, NB, D), jnp.float32),
        pltpu.VMEM((2, NB * NB), jnp.float32),
        pltpu.SemaphoreType.DMA,
        pltpu.SemaphoreType.DMA,
        pltpu.SemaphoreType.DMA,
        pltpu.SemaphoreType.DMA,
        pltpu.SemaphoreType.DMA,
        pltpu.SemaphoreType.DMA,
        pltpu.SemaphoreType.DMA,
        pltpu.SemaphoreType.DMA,
        pltpu.SemaphoreType.DMA,
    ],
)
def _sc_gather(usr_w, ent_w, relT, u, v, e1r, e2r, r2, r1,
               user_out, ev0_out, ev1_out, agg2_out, p1_out, ew_out,
               idx_v, user_rows, ev0_rows, e1_v, rowbuf, relT_v, r1_v,
               p1_v, e2_v, r2_v, rows_v, agg_v, ws_v,
               sem, semG0, semG1, semO0, semO1, semM0, semM1, semA0, semA1):
    wid = lax.axis_index("s") * NC + lax.axis_index("c")
    base = wid * BPW
    fbase = wid * BPW * NB
    semG = (semG0, semG1)
    semO = (semO0, semO1)
    semM = (semM0, semM1)
    semA = (semA0, semA1)

    pltpu.sync_copy(u.at[pl.ds(base, BPW)], idx_v)
    pltpu.async_copy(usr_w.at[idx_v], user_rows, sem).wait()
    pltpu.sync_copy(user_rows, user_out.at[pl.ds(base, BPW)])

    pltpu.sync_copy(v.at[pl.ds(base, BPW)], idx_v)
    pltpu.async_copy(ent_w.at[idx_v], ev0_rows, sem).wait()
    pltpu.sync_copy(ev0_rows, ev0_out.at[pl.ds(base, BPW)])

    pltpu.sync_copy(e1r.at[pl.ds(wid * NB, NB)], e1_v)

    def ev1_issue(c, buf):
        pltpu.async_copy(ent_w.at[e1_v.at[c]], rowbuf.at[buf], semG[buf])

    def ev1_out_copy(c, buf):
        return pltpu.make_async_copy(
            rowbuf.at[buf], ev1_out.at[pl.ds(fbase + c * BPW, BPW)], semO[buf])

    ev1_issue(0, 0)
    for c in range(NB):
        buf = c & 1
        if c + 1 < NB:
            if c - 1 >= 0:
                ev1_out_copy(c - 1, 1 - buf).wait()
            ev1_issue(c + 1, 1 - buf)
        pltpu.make_async_copy(ent_w.at[e1_v.at[c]], rowbuf.at[buf],
                              semG[buf]).wait()
        ev1_out_copy(c, buf).start()
    ev1_out_copy(NB - 2, (NB - 2) & 1).wait()
    ev1_out_copy(NB - 1, (NB - 1) & 1).wait()

    pltpu.sync_copy(relT, relT_v)
    pltpu.sync_copy(r1.at[pl.ds(base, BPW)], r1_v)
    pltpu.sync_copy(e2r.at[pl.ds(wid * 2 * BPW, 2 * BPW)], e2_v)
    pltpu.sync_copy(r2.at[pl.ds(base, BPW)], r2_v)

    def main_issue(b, buf):
        pltpu.async_copy(ent_w.at[e2_v.at[2 * b]],
                         rows_v.at[buf, pl.ds(0, BPW)], semM[buf])
        pltpu.async_copy(ent_w.at[e2_v.at[2 * b + 1]],
                         rows_v.at[buf, pl.ds(BPW, BPW)], semM[buf])

    def main_drain(b, buf):
        pltpu.make_async_copy(ent_w.at[e2_v.at[2 * b]],
                              rows_v.at[buf, pl.ds(0, BPW)], semM[buf]).wait()
        pltpu.make_async_copy(ent_w.at[e2_v.at[2 * b + 1]],
                              rows_v.at[buf, pl.ds(BPW, BPW)], semM[buf]).wait()

    def agg_copy(b, buf):
        return pltpu.make_async_copy(
            agg_v.at[buf], agg2_out.at[pl.ds(fbase + b * NB, NB)], semA[buf])

    def ew_copy(b, buf):
        return pltpu.make_async_copy(
            ws_v.at[buf], ew_out.at[base + b], semA[buf])

    def compute_b(b, buf, abuf):
        ur = [user_rows[b, 0:16], user_rows[b, 16:32]]
        sb = []
        for rc in range(4):
            accs = [jnp.zeros((16,), jnp.float32) for _ in range(4)]
            for dd in range(D):
                accs[dd % 4] = accs[dd % 4] + (
                    ur[dd // 16][dd % 16] * relT_v[dd, rc * 16:(rc + 1) * 16])
            sb.append((accs[0] + accs[1]) + (accs[2] + accs[3]))
        p1_v[b, :] = jnp.exp(_gather64(sb, r1_v[b, :]))
        ws = []
        for n in range(NB):
            e = jnp.exp(_gather64(sb, r2_v[b, n * 16:(n + 1) * 16]))
            ws_v[abuf, n * 16:(n + 1) * 16] = e
            ws.append(e)
        for n in range(NB):
            e = ws[n]
            a0s = [jnp.zeros((16,), jnp.float32) for _ in range(4)]
            a1s = [jnp.zeros((16,), jnp.float32) for _ in range(4)]
            for k in range(NB):
                w = e[k]
                a0s[k % 4] = a0s[k % 4] + w * rows_v[buf, n * NB + k, 0:16]
                a1s[k % 4] = a1s[k % 4] + w * rows_v[buf, n * NB + k, 16:32]
            agg_v[abuf, n, 0:16] = (a0s[0] + a0s[1]) + (a0s[2] + a0s[3])
            agg_v[abuf, n, 16:32] = (a1s[0] + a1s[1]) + (a1s[2] + a1s[3])
        agg_copy(b, abuf).start()
        ew_copy(b, abuf).start()

    main_issue(0, 0)

    def main_body(i, carry):
        b0 = 2 * i
        main_issue(b0 + 1, 1)
        main_drain(b0, 0)

        @pl.when(i > 0)
        def _():
            agg_copy(b0 - 2, 0).wait()
            ew_copy(b0 - 2, 0).wait()

        compute_b(b0, 0, 0)

        @pl.when(i < BPW // 2 - 1)
        def _():
            main_issue(b0 + 2, 0)

        main_drain(b0 + 1, 1)

        @pl.when(i > 0)
        def _():
            agg_copy(b0 - 1, 1).wait()
            ew_copy(b0 - 1, 1).wait()

        compute_b(b0 + 1, 1, 1)
        return carry

    lax.fori_loop(0, BPW // 2, main_body, 0)
    agg_copy(BPW - 2, 0).wait()
    ew_copy(BPW - 2, 0).wait()
    agg_copy(BPW - 1, 1).wait()
    ew_copy(BPW - 1, 1).wait()
    pltpu.sync_copy(p1_v, p1_out.at[pl.ds(base, BPW)])


CB = 8192


def _pack_body(src_ref, out_ref):
    x = src_ref[...]
    y = jnp.concatenate(
        [x[:, q * (CB // 4):(q + 1) * (CB // 4)] for q in range(4)], axis=0)
    out_ref[...] = y.T


def _pack_table(tT):
    n = tT.shape[1]
    grid = (n + CB - 1) // CB
    out = pl.pallas_call(
        _pack_body,
        grid=(grid,),
        in_specs=[pl.BlockSpec((D, CB), lambda i: (0, i))],
        out_specs=pl.BlockSpec((CB // 4, 128), lambda i: (i, 0)),
        out_shape=jax.ShapeDtypeStruct((grid * (CB // 4), 128), jnp.float32),
    )(tT)
    return out.reshape(grid * CB, D)


_CBQ = CB // 4
_CBSH = _CBQ.bit_length() - 1


def _rowmap(i):
    return (i & ~(CB - 1)) + 4 * (i & (_CBQ - 1)) + ((i >> _CBSH) & 3)


BB = 256


def _tc_body(user_ref, ev0_ref, ev1_ref, agg2_ref, p1_ref, ew_ref, W_ref,
             b_ref, Wbig_ref, H_ref, Kp_ref, G_ref, b512_ref, out_ref):
    f32 = jnp.float32
    user = user_ref[...]
    ev0 = ev0_ref[...]
    ev1 = ev1_ref[...]
    agg2 = agg2_ref[...]
    p1r = p1_ref[...]
    ew = ew_ref[...]
    W = W_ref[...]
    bias = b_ref[...]
    z = jnp.dot(ew, H_ref[...], preferred_element_type=f32)
    rzr = jnp.dot(1.0 / z, Kp_ref[...], preferred_element_type=f32)
    p1 = p1r / jnp.sum(p1r, axis=1, keepdims=True)
    P = jnp.dot(p1, Kp_ref[...], preferred_element_type=f32)
    x = ev1 + agg2 * rzr
    h1f = jax.nn.sigmoid(
        jnp.dot(x, Wbig_ref[...], preferred_element_type=f32) + b512_ref[...])
    itemagg = jnp.dot(h1f * P, G_ref[...], preferred_element_type=f32)
    agg1 = jnp.dot(ev1 * P, G_ref[...], preferred_element_type=f32)
    h0 = jax.nn.sigmoid(
        jnp.dot(ev0 + agg1, W, preferred_element_type=f32) + bias)
    item = jnp.tanh(
        jnp.dot(h0 + itemagg, W, preferred_element_type=f32) + bias)
    out_ref[...] = jax.nn.sigmoid(jnp.sum(user * item, axis=1)).reshape(1, 1, BB)


def _tc_tail(user, ev0, ev1f, agg2f, p1, ew, W, b2):
    grid = B // BB
    f32 = jnp.float32
    Wbig = jnp.kron(jnp.eye(NB, dtype=f32), W)
    H = jnp.kron(jnp.eye(NB, dtype=f32), jnp.ones((NB, 1), f32))
    Kp = jnp.kron(jnp.eye(NB, dtype=f32), jnp.ones((1, D), f32))
    G = jnp.kron(jnp.ones((NB, 1), f32), jnp.eye(D, dtype=f32))
    b512 = jnp.tile(b2, (1, NB))
    out = pl.pallas_call(
        _tc_body,
        grid=(grid,),
        in_specs=[
            pl.BlockSpec((BB, D), lambda i: (i, 0)),
            pl.BlockSpec((BB, D), lambda i: (i, 0)),
            pl.BlockSpec((BB, NB * D), lambda i: (i, 0)),
            pl.BlockSpec((BB, NB * D), lambda i: (i, 0)),
            pl.BlockSpec((BB, NB), lambda i: (i, 0)),
            pl.BlockSpec((BB, NB * NB), lambda i: (i, 0)),
            pl.BlockSpec((D, D), lambda i: (0, 0)),
            pl.BlockSpec((1, D), lambda i: (0, 0)),
            pl.BlockSpec((NB * D, NB * D), lambda i: (0, 0)),
            pl.BlockSpec((NB * NB, NB), lambda i: (0, 0)),
            pl.BlockSpec((NB, NB * D), lambda i: (0, 0)),
            pl.BlockSpec((NB * D, D), lambda i: (0, 0)),
            pl.BlockSpec((1, NB * D), lambda i: (0, 0)),
        ],
        out_specs=pl.BlockSpec((1, 1, BB), lambda i: (i, 0, 0)),
        out_shape=jax.ShapeDtypeStruct((grid, 1, BB), jnp.float32),
    )(user, ev0, ev1f, agg2f, p1, ew, W, b2, Wbig, H, Kp, G, b512)
    return out.reshape(B)


def kernel(usr_w, ent_w, rel_w, W, b, u, v, e1, e2, r1, r2):
    ent_g = _pack_table(ent_w.T)
    usr_g = _pack_table(usr_w.T)
    relT = rel_w.T
    e1r = _rowmap(e1).reshape(B * NB // BPW, BPW)
    e2r = _rowmap(e2).reshape(2 * B, BPW)
    user, ev0, ev1f, agg2f, p1, ew = _sc_gather(
        usr_g, ent_g, relT, _rowmap(u), _rowmap(v), e1r, e2r, r2, r1)
    return _tc_tail(user, ev0, ev1f.reshape(B, NB * D),
                    agg2f.reshape(B, NB * D), p1, ew, W, b.reshape(1, D))


# speedup vs baseline: 1.1670x; 1.0059x over previous
import functools

import jax
import jax.numpy as jnp
from jax import lax
from jax.experimental import pallas as pl
from jax.experimental.pallas import tpu as pltpu
from jax.experimental.pallas import tpu_sc as plsc

B = 4096
D = 32
NB = 16
NR = 64
NC = 2
NS = 16
NW = NC * NS
BPW = B // NW

_mesh = plsc.VectorSubcoreMesh(core_axis_name="c", subcore_axis_name="s")
_PIB = lax.GatherScatterMode.PROMISE_IN_BOUNDS


def _perm(x, idx):
    return jnp.take_along_axis(x, idx, axis=0, mode=_PIB)


def _gather64(sb, r):
    out = jnp.zeros((16,), jnp.float32)
    for c in range(4):
        idx = r - c * 16
        m = (idx >= 0) & (idx < 16)
        idxc = jnp.clip(idx, 0, 15)
        out = jnp.where(m, _perm(sb[c], idxc), out)
    return out


@functools.partial(
    pl.kernel,
    out_type=[
        jax.ShapeDtypeStruct((B, D), jnp.float32),
        jax.ShapeDtypeStruct((B, D), jnp.float32),
        jax.ShapeDtypeStruct((B * NB, D), jnp.float32),
        jax.ShapeDtypeStruct((B * NB, D), jnp.float32),
        jax.ShapeDtypeStruct((B, NB), jnp.float32),
        jax.ShapeDtypeStruct((B, NB * NB), jnp.float32),
    ],
    mesh=_mesh,
    compiler_params=pltpu.CompilerParams(use_tc_tiling_on_sc=False),
    scratch_types=[
        pltpu.VMEM((BPW,), jnp.int32),
        pltpu.VMEM((BPW, D), jnp.float32),
        pltpu.VMEM((BPW, D), jnp.float32),
        pltpu.VMEM((NB, BPW), jnp.int32),
        pltpu.VMEM((2, BPW, D), jnp.float32),
        pltpu.VMEM((D, NR), jnp.float32),
        pltpu.VMEM((BPW, NB), jnp.int32),
        pltpu.VMEM((BPW, NB), jnp.float32),
        pltpu.VMEM((2 * BPW, BPW), jnp.int32),
        pltpu.VMEM((BPW, NB * NB), jnp.int32),
        pltpu.VMEM((2, NB * NB, D), jnp.float32),
        pltpu.VMEM((2, NB, D), jnp.float32),
        pltpu.VMEM((2, NB * NB), jnp.float32),
        pltpu.SemaphoreType.DMA,
        pltpu.SemaphoreType.DMA,
        pltpu.SemaphoreType.DMA,
        pltpu.SemaphoreType.DMA,
        pltpu.SemaphoreType.DMA,
        pltpu.SemaphoreType.DMA,
        pltpu.SemaphoreType.DMA,
        pltpu.SemaphoreType.DMA,
        pltpu.SemaphoreType.DMA,
    ],
)
def _sc_gather(usr_w, ent_w, relT, u, v, e1r, e2r, r2, r1,
               user_out, ev0_out, ev1_out, agg2_out, p1_out, ew_out,
               idx_v, user_rows, ev0_rows, e1_v, rowbuf, relT_v, r1_v,
               p1_v, e2_v, r2_v, rows_v, agg_v, ws_v,
               sem, semG0, semG1, semO0, semO1, semM0, semM1, semA0, semA1):
    wid = lax.axis_index("s") * NC + lax.axis_index("c")
    base = wid * BPW
    fbase = wid * BPW * NB
    semG = (semG0, semG1)
    semO = (semO0, semO1)
    semM = (semM0, semM1)
    semA = (semA0, semA1)

    pltpu.sync_copy(u.at[pl.ds(base, BPW)], idx_v)
    pltpu.async_copy(usr_w.at[idx_v], user_rows, sem).wait()
    pltpu.sync_copy(user_rows, user_out.at[pl.ds(base, BPW)])

    pltpu.sync_copy(v.at[pl.ds(base, BPW)], idx_v)
    pltpu.async_copy(ent_w.at[idx_v], ev0_rows, sem).wait()
    pltpu.sync_copy(ev0_rows, ev0_out.at[pl.ds(base, BPW)])

    pltpu.sync_copy(e1r.at[pl.ds(wid * NB, NB)], e1_v)

    def ev1_issue(c, buf):
        pltpu.async_copy(ent_w.at[e1_v.at[c]], rowbuf.at[buf], semG[buf])

    def ev1_out_copy(c, buf):
        return pltpu.make_async_copy(
            rowbuf.at[buf], ev1_out.at[pl.ds(fbase + c * BPW, BPW)], semO[buf])

    ev1_issue(0, 0)
    for c in range(NB):
        buf = c & 1
        if c + 1 < NB:
            if c - 1 >= 0:
                ev1_out_copy(c - 1, 1 - buf).wait()
            ev1_issue(c + 1, 1 - buf)
        pltpu.make_async_copy(ent_w.at[e1_v.at[c]], rowbuf.at[buf],
                              semG[buf]).wait()
        ev1_out_copy(c, buf).start()
    ev1_out_copy(NB - 2, (NB - 2) & 1).wait()
    ev1_out_copy(NB - 1, (NB - 1) & 1).wait()

    pltpu.sync_copy(relT, relT_v)
    pltpu.sync_copy(r1.at[pl.ds(base, BPW)], r1_v)
    pltpu.sync_copy(e2r.at[pl.ds(wid * 2 * BPW, 2 * BPW)], e2_v)
    pltpu.sync_copy(r2.at[pl.ds(base, BPW)], r2_v)

    def main_issue(b, buf):
        pltpu.async_copy(ent_w.at[e2_v.at[2 * b]],
                         rows_v.at[buf, pl.ds(0, BPW)], semM[buf])
        pltpu.async_copy(ent_w.at[e2_v.at[2 * b + 1]],
                         rows_v.at[buf, pl.ds(BPW, BPW)], semM[buf])

    def main_drain(b, buf):
        pltpu.make_async_copy(ent_w.at[e2_v.at[2 * b]],
                              rows_v.at[buf, pl.ds(0, BPW)], semM[buf]).wait()
        pltpu.make_async_copy(ent_w.at[e2_v.at[2 * b + 1]],
                              rows_v.at[buf, pl.ds(BPW, BPW)], semM[buf]).wait()

    def agg_copy(b, buf):
        return pltpu.make_async_copy(
            agg_v.at[buf], agg2_out.at[pl.ds(fbase + b * NB, NB)], semA[buf])

    def ew_copy(b, buf):
        return pltpu.make_async_copy(
            ws_v.at[buf], ew_out.at[base + b], semA[buf])

    def compute_b(b, buf, abuf):
        ur = [user_rows[b, 0:16], user_rows[b, 16:32]]
        sb = []
        for rc in range(4):
            accs = [jnp.zeros((16,), jnp.float32) for _ in range(4)]
            for dd in range(D):
                accs[dd % 4] = accs[dd % 4] + (
                    ur[dd // 16][dd % 16] * relT_v[dd, rc * 16:(rc + 1) * 16])
            sb.append((accs[0] + accs[1]) + (accs[2] + accs[3]))
        p1_v[b, :] = jnp.exp(_gather64(sb, r1_v[b, :]))
        ws = []
        for n in range(NB):
            e = jnp.exp(_gather64(sb, r2_v[b, n * 16:(n + 1) * 16]))
            ws_v[abuf, n * 16:(n + 1) * 16] = e
            ws.append(e)
        for n in range(NB):
            e = ws[n]
            a0s = [jnp.zeros((16,), jnp.float32) for _ in range(4)]
            a1s = [jnp.zeros((16,), jnp.float32) for _ in range(4)]
            for k in range(NB):
                w = e[k]
                a0s[k % 4] = a0s[k % 4] + w * rows_v[buf, n * NB + k, 0:16]
                a1s[k % 4] = a1s[k % 4] + w * rows_v[buf, n * NB + k, 16:32]
            agg_v[abuf, n, 0:16] = (a0s[0] + a0s[1]) + (a0s[2] + a0s[3])
            agg_v[abuf, n, 16:32] = (a1s[0] + a1s[1]) + (a1s[2] + a1s[3])
        agg_copy(b, abuf).start()
        ew_copy(b, abuf).start()

    main_issue(0, 0)

    def main_body(i, carry):
        b0 = 2 * i
        main_issue(b0 + 1, 1)
        main_drain(b0, 0)

        @pl.when(i > 0)
        def _():
            agg_copy(b0 - 2, 0).wait()
            ew_copy(b0 - 2, 0).wait()

        compute_b(b0, 0, 0)

        @pl.when(i < BPW // 2 - 1)
        def _():
            main_issue(b0 + 2, 0)

        main_drain(b0 + 1, 1)

        @pl.when(i > 0)
        def _():
            agg_copy(b0 - 1, 1).wait()
            ew_copy(b0 - 1, 1).wait()

        compute_b(b0 + 1, 1, 1)
        return carry

    lax.fori_loop(0, BPW // 2, main_body, 0)
    agg_copy(BPW - 2, 0).wait()
    ew_copy(BPW - 2, 0).wait()
    agg_copy(BPW - 1, 1).wait()
    ew_copy(BPW - 1, 1).wait()
    pltpu.sync_copy(p1_v, p1_out.at[pl.ds(base, BPW)])


CB = 8192


def _pack_body(src_ref, out_ref):
    x = src_ref[...]
    y = jnp.concatenate(
        [x[:, q * (CB // 4):(q + 1) * (CB // 4)] for q in range(4)], axis=0)
    out_ref[...] = y.T


def _pack_table(tT):
    n = tT.shape[1]
    grid = (n + CB - 1) // CB
    out = pl.pallas_call(
        _pack_body,
        grid=(grid,),
        in_specs=[pl.BlockSpec((D, CB), lambda i: (0, i))],
        out_specs=pl.BlockSpec((CB // 4, 128), lambda i: (i, 0)),
        out_shape=jax.ShapeDtypeStruct((grid * (CB // 4), 128), jnp.float32),
    )(tT)
    return out.reshape(grid * CB, D)


_CBQ = CB // 4
_CBSH = _CBQ.bit_length() - 1


def _rowmap(i):
    return (i & ~(CB - 1)) + 4 * (i & (_CBQ - 1)) + ((i >> _CBSH) & 3)


BB = 256


def _tc_body(user_ref, ev0_ref, ev1_ref, agg2_ref, p1_ref, ew_ref, W_ref,
             b_ref, Wbig_ref, H_ref, Kp_ref, G_ref, b512_ref, out_ref):
    f32 = jnp.float32
    user = user_ref[...]
    ev0 = ev0_ref[...]
    ev1 = ev1_ref[...]
    agg2 = agg2_ref[...]
    p1r = p1_ref[...]
    ew = ew_ref[...]
    W = W_ref[...]
    bias = b_ref[...]
    z = jnp.dot(ew, H_ref[...], preferred_element_type=f32)
    rzr = jnp.dot(1.0 / z, Kp_ref[...], preferred_element_type=f32)
    p1 = p1r / jnp.sum(p1r, axis=1, keepdims=True)
    P = jnp.dot(p1, Kp_ref[...], preferred_element_type=f32)
    x = ev1 + agg2 * rzr
    h1f = jax.nn.sigmoid(
        jnp.dot(x, Wbig_ref[...], preferred_element_type=f32) + b512_ref[...])
    itemagg = jnp.dot(h1f * P, G_ref[...], preferred_element_type=f32)
    agg1 = jnp.dot(ev1 * P, G_ref[...], preferred_element_type=f32)
    h0 = jax.nn.sigmoid(
        jnp.dot(ev0 + agg1, W, preferred_element_type=f32) + bias)
    item = jnp.tanh(
        jnp.dot(h0 + itemagg, W, preferred_element_type=f32) + bias)
    out_ref[...] = jax.nn.sigmoid(jnp.sum(user * item, axis=1)).reshape(1, 1, BB)


def _tc_tail(user, ev0, ev1f, agg2f, p1, ew, W, b2):
    grid = B // BB
    f32 = jnp.float32
    Wbig = jnp.kron(jnp.eye(NB, dtype=f32), W)
    H = jnp.kron(jnp.eye(NB, dtype=f32), jnp.ones((NB, 1), f32))
    Kp = jnp.kron(jnp.eye(NB, dtype=f32), jnp.ones((1, D), f32))
    G = jnp.kron(jnp.ones((NB, 1), f32), jnp.eye(D, dtype=f32))
    b512 = jnp.tile(b2, (1, NB))
    out = pl.pallas_call(
        _tc_body,
        grid=(grid,),
        in_specs=[
            pl.BlockSpec((BB, D), lambda i: (i, 0)),
            pl.BlockSpec((BB, D), lambda i: (i, 0)),
            pl.BlockSpec((BB, NB * D), lambda i: (i, 0)),
            pl.BlockSpec((BB, NB * D), lambda i: (i, 0)),
            pl.BlockSpec((BB, NB), lambda i: (i, 0)),
            pl.BlockSpec((BB, NB * NB), lambda i: (i, 0)),
            pl.BlockSpec((D, D), lambda i: (0, 0)),
            pl.BlockSpec((1, D), lambda i: (0, 0)),
            pl.BlockSpec((NB * D, NB * D), lambda i: (0, 0)),
            pl.BlockSpec((NB * NB, NB), lambda i: (0, 0)),
            pl.BlockSpec((NB, NB * D), lambda i: (0, 0)),
            pl.BlockSpec((NB * D, D), lambda i: (0, 0)),
            pl.BlockSpec((1, NB * D), lambda i: (0, 0)),
        ],
        out_specs=pl.BlockSpec((1, 1, BB), lambda i: (i, 0, 0)),
        out_shape=jax.ShapeDtypeStruct((grid, 1, BB), jnp.float32),
    )(user, ev0, ev1f, agg2f, p1, ew, W, b2, Wbig, H, Kp, G, b512)
    return out.reshape(B)


def kernel(usr_w, ent_w, rel_w, W, b, u, v, e1, e2, r1, r2):
    ent_g = _pack_table(ent_w.T)
    usr_g = _pack_table(usr_w.T)
    relT = rel_w.T
    e1r = _rowmap(e1).reshape(B * NB // BPW, BPW)
    e2r = _rowmap(e2).reshape(2 * B, BPW)
    user, ev0, ev1f, agg2f, p1, ew = _sc_gather(
        usr_g, ent_g, relT, _rowmap(u), _rowmap(v), e1r, e2r, r2, r1)
    return _tc_tail(user, ev0, ev1f.reshape(B, NB * D),
                    agg2f.reshape(B, NB * D), p1, ew, W, b.reshape(1, D))
